# baseline (speedup 1.0000x reference)
import functools

import jax
import jax.numpy as jnp
import numpy as np
from jax import lax
from jax.experimental import pallas as pl
from jax.experimental.pallas import tpu as pltpu
from jax.experimental.pallas import tpu_sc as plsc

_N = 10000
_D = 128
_K = 3
_NC = 2
_NS = 16
_BATCH = 128
_HALF = 64
_CHK = 24
_NZ = _N + 112
_R = 2000
_NBLK = _N // _R
_GBYTES = _BATCH * (_D // 2) * 4
_SBYTES = _HALF * _D * 4

_PERM = np.concatenate([
    np.concatenate([32 * j + 2 * np.arange(16),
                    32 * j + 2 * np.arange(16) + 1])
    for j in range(_D // 32)
])


def _mlp2(a, W0, b0, W1, b1):
    dn = (((1,), (0,)), ((), ()))
    h = lax.dot_general(a, W0, dn, precision=lax.Precision.HIGHEST,
                        preferred_element_type=jnp.float32) + b0
    h = jnp.maximum(h, 0.0)
    return lax.dot_general(h, W1, dn, precision=lax.Precision.HIGHEST,
                           preferred_element_type=jnp.float32) + b1


def _init_body(x_r, W0a, W0b, b0a, b0b, Wpa, Wpb, bpa, bpb, st_r, T_r):
    st = _mlp2(x_r[...], W0a[...], b0a[...], W0b[...], b0b[...])
    st_r[...] = st
    T_r[0] = jnp.maximum(_mlp2(st, Wpa[0], bpa[0], Wpb[0], bpb[0]),
                         0.0).astype(jnp.bfloat16)


def _masked_update(z, st, Wua, Wub, bua, bub):
    c = pl.program_id(0)
    i = pl.program_id(1)
    u = jnp.maximum(_mlp2(z, Wua, bua, Wub, bub), 0.0)
    row = i * _R + lax.broadcasted_iota(jnp.int32, (_R, _D), 0)
    sink = jnp.where(c == 0, _N - 1, 0)
    return jnp.where(row == sink, 0.0, u) + st


def _mid_body(z_r, st_r, Wua, Wub, bua, bub, Wpa, Wpb, bpa, bpb, T_r):
    y = _masked_update(z_r[0], st_r[...], Wua[0], Wub[0], bua[0], bub[0])
    T_r[0] = jnp.maximum(_mlp2(y, Wpa[0], bpa[0], Wpb[0], bpb[0]),
                         0.0).astype(jnp.bfloat16)


def _fin_body(z_r, st_r, Wua, Wub, bua, bub, out_r):
    out_r[...] = _masked_update(z_r[0], st_r[...],
                                Wua[0], Wub[0], bua[0], bub[0])


def _full2(shape):
    return pl.BlockSpec(shape, lambda c, i: (0, 0))


def _stk3(shape):
    return pl.BlockSpec(shape, lambda c, i: (c, 0, 0))


def _build_tc_calls(interpret=False):
    rows = pl.BlockSpec((_R, _D), lambda c, i: (i, 0))
    rows3 = pl.BlockSpec((1, _R, _D), lambda c, i: (c, i, 0))
    w = _full2((_D, _D))
    b = _full2((1, _D))
    w3 = _stk3((1, _D, _D))
    b3 = _stk3((1, 1, _D))

    init = pl.pallas_call(
        _init_body,
        grid=(_NC, _NBLK),
        in_specs=[rows, w, w, b, b, w3, w3, b3, b3],
        out_specs=[rows, rows3],
        out_shape=[jax.ShapeDtypeStruct((_N, _D), jnp.float32),
                   jax.ShapeDtypeStruct((_NC, _N, _D), jnp.bfloat16)],
        interpret=interpret,
    )
    mid = pl.pallas_call(
        _mid_body,
        grid=(_NC, _NBLK),
        in_specs=[rows3, rows, w3, w3, b3, b3, w3, w3, b3, b3],
        out_specs=rows3,
        out_shape=jax.ShapeDtypeStruct((_NC, _N, _D), jnp.bfloat16),
        interpret=interpret,
    )
    fin = pl.pallas_call(
        _fin_body,
        grid=(_NC, _NBLK),
        in_specs=[rows3, rows, w3, w3, b3, b3],
        out_specs=pl.BlockSpec((_R, _D), lambda c, i: (i, c)),
        out_shape=jax.ShapeDtypeStruct((_N, 2 * _D), jnp.float32),
        interpret=interpret,
    )
    return init, mid, fin


_init_call, _mid_call, _fin_call = _build_tc_calls()


def _sc_segment(T2, gidx, sidx, nb):
    mesh = plsc.VectorSubcoreMesh(core_axis_name="c", subcore_axis_name="s")
    nchunk = nb // _CHK
    zrows = _NZ // _NS

    @functools.partial(
        pl.kernel,
        out_type=jax.ShapeDtypeStruct((_NC, _NZ, _D), jnp.float32),
        mesh=mesh,
        compiler_params=pltpu.CompilerParams(use_tc_tiling_on_sc=False),
        scratch_types=[
            pltpu.VMEM((_CHK, _BATCH), jnp.int32),
            pltpu.VMEM((2 * _CHK, _HALF), jnp.int32),
            pltpu.VMEM((3, _BATCH, _D // 2), jnp.int32),
            pltpu.VMEM((2, _HALF, _D), jnp.float32),
            pltpu.VMEM_SHARED((_NZ, _D), jnp.float32),
            pltpu.SemaphoreType.DMA,
            pltpu.SemaphoreType.DMA,
        ],
    )
    def k(T_hbm, g_hbm, s_hbm, out_hbm, g_v, s_v, b16, f32b, z_sh,
          gsem, ssem):
        c = lax.axis_index("c")
        s = lax.axis_index("s")
        wid = c * _NS + s

        zbuf = f32b.at[0]

        def _zb(t, carry):
            zbuf[lax.div(t, 8), pl.ds(lax.rem(t, 8) * 16, 16)] = (
                jnp.zeros((16,), jnp.float32))
            return carry

        lax.fori_loop(0, _HALF * 8, _zb, 0)

        zb0 = s * zrows
        nfull = zrows // _HALF
        for j in range(nfull):
            pltpu.sync_copy(zbuf, z_sh.at[pl.ds(zb0 + j * _HALF, _HALF)])
        rem = zrows - nfull * _HALF
        if rem:
            pltpu.sync_copy(zbuf.at[pl.ds(0, rem)],
                            z_sh.at[pl.ds(zb0 + zrows - rem, rem)])

        plsc.subcore_barrier()

        for _ in range(2):
            pltpu.async_copy(zbuf, z_sh.at[pl.ds(_N, _HALF)], ssem)

        def _chunk(co, carry):
            pltpu.sync_copy(g_hbm.at[wid].at[pl.ds(co * _CHK, _CHK)], g_v)
            pltpu.sync_copy(
                s_hbm.at[wid].at[pl.ds(co * 2 * _CHK, 2 * _CHK)], s_v)
            for slot in range(3):
                pltpu.async_copy(
                    T_hbm.at[g_v.at[slot]], b16.at[slot], gsem)

            def _trip(p, inner):
                for slot in range(3):
                    bloc = 3 * p + slot
                    pltpu.make_async_copy(
                        T_hbm.at[pl.ds(0, _BATCH)], b16.at[slot],
                        gsem).wait()
                    for h in range(2):
                        pltpu.make_async_copy(
                            out_hbm.at[c].at[pl.ds(0, _HALF)], f32b.at[h],
                            ssem).wait()

                        def _cv(r, carry2, _slot=slot, _h=h):
                            for q in range(4):
                                v = b16[_slot, _h * _HALF + r,
                                        pl.ds(q * 16, 16)]
                                ev = lax.bitcast_convert_type(
                                    lax.shift_left(v, 16), jnp.float32)
                                od = lax.bitcast_convert_type(
                                    jnp.bitwise_and(v, jnp.int32(-65536)),
                                    jnp.float32)
                                f32b[_h, r, pl.ds(q * 32, 16)] = ev
                                f32b[_h, r, pl.ds(q * 32 + 16, 16)] = od
                            return carry2

                        lax.fori_loop(0, _HALF, _cv, 0, unroll=4)
                        pltpu.async_copy(
                            f32b.at[h], z_sh.at[s_v.at[2 * bloc + h]],
                            ssem, add=True)

                    @pl.when(p < _CHK // 3 - 1)
                    def _():
                        pltpu.async_copy(
                            T_hbm.at[g_v.at[bloc + 3]], b16.at[slot], gsem)

                return inner

            lax.fori_loop(0, _CHK // 3, _trip, 0)
            return carry

        lax.fori_loop(0, nchunk, _chunk, 0)

        for h in range(2):
            pltpu.make_async_copy(
                out_hbm.at[c].at[pl.ds(0, _HALF)], f32b.at[h], ssem).wait()

        plsc.subcore_barrier()

        pltpu.sync_copy(z_sh.at[pl.ds(zb0, zrows)],
                        out_hbm.at[c].at[pl.ds(zb0, zrows)])

    return k(T2, gidx, sidx)


def _prep_indices(edge_index):
    E = edge_index.shape[1]
    nb = -(-E // (_NS * _BATCH * _CHK)) * _CHK
    cap = _NS * nb * _BATCH
    pad = cap - E

    src = edge_index[0].astype(jnp.int32)
    dst = edge_index[1].astype(jnp.int32)
    pz = jnp.zeros((pad,), jnp.int32)
    pr = jnp.full((pad,), _N, jnp.int32)
    gidx = jnp.concatenate([src, pz, dst + _N, pz]).reshape(
        _NC * _NS, nb, _BATCH)
    sidx = jnp.concatenate([dst, pr, src, pr]).reshape(
        _NC * _NS, 2 * nb, _HALF)
    return gidx, sidx, nb


def _prep_weights(Ws, bs):
    perm = jnp.asarray(_PERM)
    W0a, W0b = Ws[0, 0], Ws[0, 1]
    b0a = bs[0, 0].reshape(1, _D)
    b0b = bs[0, 1].reshape(1, _D)
    Wpa = jnp.stack([Ws[1, 0], Ws[3, 0]])
    Wpb = jnp.stack([Ws[1, 1], Ws[3, 1]])
    bpa = jnp.stack([bs[1, 0], bs[3, 0]])[:, None, :]
    bpb = jnp.stack([bs[1, 1], bs[3, 1]])[:, None, :]
    Wua = jnp.stack([Ws[2, 0], Ws[4, 0]])[:, perm, :]
    Wub = jnp.stack([Ws[2, 1], Ws[4, 1]])
    bua = jnp.stack([bs[2, 0], bs[4, 0]])[:, None, :]
    bub = jnp.stack([bs[2, 1], bs[4, 1]])[:, None, :]
    return (W0a, W0b, b0a, b0b, Wpa, Wpb, bpa, bpb,
            Wua, Wub, bua, bub)


def kernel(x, edge_index, Ws, bs):
    gidx, sidx, nb = _prep_indices(edge_index)
    (W0a, W0b, b0a, b0b, Wpa, Wpb, bpa, bpb,
     Wua, Wub, bua, bub) = _prep_weights(Ws, bs)

    st, T = _init_call(x, W0a, W0b, b0a, b0b, Wpa, Wpb, bpa, bpb)
    out = None
    for step in range(_K):
        Tp = lax.bitcast_convert_type(
            T.reshape(_NC * _N, _D // 2, 2), jnp.int32)
        zp = _sc_segment(Tp, gidx, sidx, nb)
        if step < _K - 1:
            T = _mid_call(zp, st, Wua, Wub, bua, bub, Wpa, Wpb, bpa, bpb)
        else:
            out = _fin_call(zp, st, Wua, Wub, bua, bub)
    return out


# speedup vs baseline: 1.0005x; 1.0005x over previous
import functools

import jax
import jax.numpy as jnp
import numpy as np
from jax import lax
from jax.experimental import pallas as pl
from jax.experimental.pallas import tpu as pltpu
from jax.experimental.pallas import tpu_sc as plsc

_N = 10000
_D = 128
_K = 3
_NC = 2
_NS = 16
_BATCH = 128
_HALF = 64
_CHK = 12
_NZ = _N + 112
_R = 2000
_NBLK = _N // _R
_GBYTES = _BATCH * (_D // 2) * 4
_SBYTES = _HALF * _D * 4

_PERM = np.concatenate([
    np.concatenate([32 * j + 2 * np.arange(16),
                    32 * j + 2 * np.arange(16) + 1])
    for j in range(_D // 32)
])


def _mlp2(a, W0, b0, W1, b1):
    dn = (((1,), (0,)), ((), ()))
    h = lax.dot_general(a, W0, dn, precision=lax.Precision.HIGHEST,
                        preferred_element_type=jnp.float32) + b0
    h = jnp.maximum(h, 0.0)
    return lax.dot_general(h, W1, dn, precision=lax.Precision.HIGHEST,
                           preferred_element_type=jnp.float32) + b1


def _init_body(x_r, W0a, W0b, b0a, b0b, Wpa, Wpb, bpa, bpb, st_r, T_r):
    st = _mlp2(x_r[...], W0a[...], b0a[...], W0b[...], b0b[...])
    st_r[...] = st
    T_r[0] = jnp.maximum(_mlp2(st, Wpa[0], bpa[0], Wpb[0], bpb[0]),
                         0.0).astype(jnp.bfloat16)


def _masked_update(z, st, Wua, Wub, bua, bub):
    c = pl.program_id(0)
    i = pl.program_id(1)
    u = jnp.maximum(_mlp2(z, Wua, bua, Wub, bub), 0.0)
    row = i * _R + lax.broadcasted_iota(jnp.int32, (_R, _D), 0)
    sink = jnp.where(c == 0, _N - 1, 0)
    return jnp.where(row == sink, 0.0, u) + st


def _mid_body(z_r, st_r, Wua, Wub, bua, bub, Wpa, Wpb, bpa, bpb, T_r):
    y = _masked_update(z_r[0], st_r[...], Wua[0], Wub[0], bua[0], bub[0])
    T_r[0] = jnp.maximum(_mlp2(y, Wpa[0], bpa[0], Wpb[0], bpb[0]),
                         0.0).astype(jnp.bfloat16)


def _fin_body(z_r, st_r, Wua, Wub, bua, bub, out_r):
    out_r[...] = _masked_update(z_r[0], st_r[...],
                                Wua[0], Wub[0], bua[0], bub[0])


def _full2(shape):
    return pl.BlockSpec(shape, lambda c, i: (0, 0))


def _stk3(shape):
    return pl.BlockSpec(shape, lambda c, i: (c, 0, 0))


def _build_tc_calls(interpret=False):
    rows = pl.BlockSpec((_R, _D), lambda c, i: (i, 0))
    rows3 = pl.BlockSpec((1, _R, _D), lambda c, i: (c, i, 0))
    w = _full2((_D, _D))
    b = _full2((1, _D))
    w3 = _stk3((1, _D, _D))
    b3 = _stk3((1, 1, _D))

    init = pl.pallas_call(
        _init_body,
        grid=(_NC, _NBLK),
        in_specs=[rows, w, w, b, b, w3, w3, b3, b3],
        out_specs=[rows, rows3],
        out_shape=[jax.ShapeDtypeStruct((_N, _D), jnp.float32),
                   jax.ShapeDtypeStruct((_NC, _N, _D), jnp.bfloat16)],
        interpret=interpret,
    )
    mid = pl.pallas_call(
        _mid_body,
        grid=(_NC, _NBLK),
        in_specs=[rows3, rows, w3, w3, b3, b3, w3, w3, b3, b3],
        out_specs=rows3,
        out_shape=jax.ShapeDtypeStruct((_NC, _N, _D), jnp.bfloat16),
        interpret=interpret,
    )
    fin = pl.pallas_call(
        _fin_body,
        grid=(_NC, _NBLK),
        in_specs=[rows3, rows, w3, w3, b3, b3],
        out_specs=pl.BlockSpec((_R, _D), lambda c, i: (i, c)),
        out_shape=jax.ShapeDtypeStruct((_N, 2 * _D), jnp.float32),
        interpret=interpret,
    )
    return init, mid, fin


_init_call, _mid_call, _fin_call = _build_tc_calls()


def _sc_segment(T2, gidx, sidx, nb):
    mesh = plsc.VectorSubcoreMesh(core_axis_name="c", subcore_axis_name="s")
    nchunk = nb // _CHK
    zrows = _NZ // _NS

    @functools.partial(
        pl.kernel,
        out_type=jax.ShapeDtypeStruct((_NC, _NZ, _D), jnp.float32),
        mesh=mesh,
        compiler_params=pltpu.CompilerParams(use_tc_tiling_on_sc=False),
        scratch_types=[
            pltpu.VMEM((_CHK, _BATCH), jnp.int32),
            pltpu.VMEM((2 * _CHK, _HALF), jnp.int32),
            pltpu.VMEM((3, _BATCH, _D // 2), jnp.int32),
            pltpu.VMEM((2, _HALF, _D), jnp.float32),
            pltpu.VMEM_SHARED((_NZ, _D), jnp.float32),
            pltpu.SemaphoreType.DMA,
            pltpu.SemaphoreType.DMA,
        ],
    )
    def k(T_hbm, g_hbm, s_hbm, out_hbm, g_v, s_v, b16, f32b, z_sh,
          gsem, ssem):
        c = lax.axis_index("c")
        s = lax.axis_index("s")
        wid = c * _NS + s

        zbuf = f32b.at[0]

        def _zb(t, carry):
            zbuf[lax.div(t, 8), pl.ds(lax.rem(t, 8) * 16, 16)] = (
                jnp.zeros((16,), jnp.float32))
            return carry

        lax.fori_loop(0, _HALF * 8, _zb, 0)

        zb0 = s * zrows
        nfull = zrows // _HALF
        for j in range(nfull):
            pltpu.sync_copy(zbuf, z_sh.at[pl.ds(zb0 + j * _HALF, _HALF)])
        rem = zrows - nfull * _HALF
        if rem:
            pltpu.sync_copy(zbuf.at[pl.ds(0, rem)],
                            z_sh.at[pl.ds(zb0 + zrows - rem, rem)])

        plsc.subcore_barrier()

        for _ in range(2):
            pltpu.async_copy(zbuf, z_sh.at[pl.ds(_N, _HALF)], ssem)

        def _chunk(co, carry):
            pltpu.sync_copy(g_hbm.at[wid].at[pl.ds(co * _CHK, _CHK)], g_v)
            pltpu.sync_copy(
                s_hbm.at[wid].at[pl.ds(co * 2 * _CHK, 2 * _CHK)], s_v)
            for slot in range(3):
                pltpu.async_copy(
                    T_hbm.at[g_v.at[slot]], b16.at[slot], gsem)

            def _pair(p, inner):
                for slot in range(3):
                    bloc = 3 * p + slot
                    pltpu.make_async_copy(
                        T_hbm.at[pl.ds(0, _BATCH)], b16.at[slot],
                        gsem).wait()
                    for h in range(2):
                        pltpu.make_async_copy(
                            out_hbm.at[c].at[pl.ds(0, _HALF)], f32b.at[h],
                            ssem).wait()

                        def _cv(t, carry2, _slot=slot, _h=h):
                            r = _h * _HALF + lax.div(t, 4)
                            q = lax.rem(t, 4)
                            v = b16[_slot, r, pl.ds(q * 16, 16)]
                            ev = lax.bitcast_convert_type(
                                lax.shift_left(v, 16), jnp.float32)
                            od = lax.bitcast_convert_type(
                                jnp.bitwise_and(v, jnp.int32(-65536)),
                                jnp.float32)
                            rr = lax.rem(r, _HALF)
                            f32b[_h, rr, pl.ds(q * 32, 16)] = ev
                            f32b[_h, rr, pl.ds(q * 32 + 16, 16)] = od
                            return carry2

                        lax.fori_loop(0, _HALF * 4, _cv, 0, unroll=8)
                        pltpu.async_copy(
                            f32b.at[h], z_sh.at[s_v.at[2 * bloc + h]],
                            ssem, add=True)

                    @pl.when(p < _CHK // 3 - 1)
                    def _():
                        pltpu.async_copy(
                            T_hbm.at[g_v.at[bloc + 3]], b16.at[slot], gsem)

                return inner

            lax.fori_loop(0, _CHK // 3, _pair, 0)
            return carry

        lax.fori_loop(0, nchunk, _chunk, 0)

        for h in range(2):
            pltpu.make_async_copy(
                out_hbm.at[c].at[pl.ds(0, _HALF)], f32b.at[h], ssem).wait()

        plsc.subcore_barrier()

        pltpu.sync_copy(z_sh.at[pl.ds(zb0, zrows)],
                        out_hbm.at[c].at[pl.ds(zb0, zrows)])

    return k(T2, gidx, sidx)


def _prep_indices(edge_index):
    E = edge_index.shape[1]
    nb = -(-E // (_NS * _BATCH * _CHK)) * _CHK
    cap = _NS * nb * _BATCH
    pad = cap - E

    src = edge_index[0].astype(jnp.int32)
    dst = edge_index[1].astype(jnp.int32)
    pz = jnp.zeros((pad,), jnp.int32)
    pr = jnp.full((pad,), _N, jnp.int32)
    gidx = jnp.concatenate([src, pz, dst + _N, pz]).reshape(
        _NC * _NS, nb, _BATCH)
    sidx = jnp.concatenate([dst, pr, src, pr]).reshape(
        _NC * _NS, 2 * nb, _HALF)
    return gidx, sidx, nb


def _prep_weights(Ws, bs):
    perm = jnp.asarray(_PERM)
    W0a, W0b = Ws[0, 0], Ws[0, 1]
    b0a = bs[0, 0].reshape(1, _D)
    b0b = bs[0, 1].reshape(1, _D)
    Wpa = jnp.stack([Ws[1, 0], Ws[3, 0]])
    Wpb = jnp.stack([Ws[1, 1], Ws[3, 1]])
    bpa = jnp.stack([bs[1, 0], bs[3, 0]])[:, None, :]
    bpb = jnp.stack([bs[1, 1], bs[3, 1]])[:, None, :]
    Wua = jnp.stack([Ws[2, 0], Ws[4, 0]])[:, perm, :]
    Wub = jnp.stack([Ws[2, 1], Ws[4, 1]])
    bua = jnp.stack([bs[2, 0], bs[4, 0]])[:, None, :]
    bub = jnp.stack([bs[2, 1], bs[4, 1]])[:, None, :]
    return (W0a, W0b, b0a, b0b, Wpa, Wpb, bpa, bpb,
            Wua, Wub, bua, bub)


def kernel(x, edge_index, Ws, bs):
    gidx, sidx, nb = _prep_indices(edge_index)
    (W0a, W0b, b0a, b0b, Wpa, Wpb, bpa, bpb,
     Wua, Wub, bua, bub) = _prep_weights(Ws, bs)

    st, T = _init_call(x, W0a, W0b, b0a, b0b, Wpa, Wpb, bpa, bpb)
    out = None
    for step in range(_K):
        Tp = lax.bitcast_convert_type(
            T.reshape(_NC * _N, _D // 2, 2), jnp.int32)
        zp = _sc_segment(Tp, gidx, sidx, nb)
        if step < _K - 1:
            T = _mid_call(zp, st, Wua, Wub, bua, bub, Wpa, Wpb, bpa, bpb)
        else:
            out = _fin_call(zp, st, Wua, Wub, bua, bub)
    return out


# speedup vs baseline: 1.5168x; 1.5160x over previous
import functools

import jax
import jax.numpy as jnp
import numpy as np
from jax import lax
from jax.experimental import pallas as pl
from jax.experimental.pallas import tpu as pltpu
from jax.experimental.pallas import tpu_sc as plsc

_N = 10000
_D = 128
_K = 3
_NC = 2
_NS = 16
_BATCH = 128
_HALF = 64
_CHK = 16
_NZ = _N + 112
_R = 2000
_NBLK = _N // _R
_GBYTES = _BATCH * (_D // 2) * 4
_SBYTES = _HALF * _D * 4

_PERM = np.concatenate([
    np.concatenate([32 * j + 2 * np.arange(16),
                    32 * j + 2 * np.arange(16) + 1])
    for j in range(_D // 32)
])


def _mlp2(a, W0, b0, W1, b1):
    dn = (((1,), (0,)), ((), ()))
    h = lax.dot_general(a, W0, dn, precision=lax.Precision.HIGHEST,
                        preferred_element_type=jnp.float32) + b0
    h = jnp.maximum(h, 0.0)
    return lax.dot_general(h, W1, dn, precision=lax.Precision.HIGHEST,
                           preferred_element_type=jnp.float32) + b1


def _init_body(x_r, W0a, W0b, b0a, b0b, Wpa, Wpb, bpa, bpb, st_r, T_r):
    st = _mlp2(x_r[...], W0a[...], b0a[...], W0b[...], b0b[...])
    st_r[...] = st
    T_r[0] = jnp.maximum(_mlp2(st, Wpa[0], bpa[0], Wpb[0], bpb[0]),
                         0.0).astype(jnp.bfloat16)


def _masked_update(z, st, Wua, Wub, bua, bub):
    c = pl.program_id(0)
    i = pl.program_id(1)
    u = jnp.maximum(_mlp2(z, Wua, bua, Wub, bub), 0.0)
    row = i * _R + lax.broadcasted_iota(jnp.int32, (_R, _D), 0)
    sink = jnp.where(c == 0, _N - 1, 0)
    return jnp.where(row == sink, 0.0, u) + st


def _mid_body(z_r, st_r, Wua, Wub, bua, bub, Wpa, Wpb, bpa, bpb, T_r):
    y = _masked_update(z_r[0], st_r[...], Wua[0], Wub[0], bua[0], bub[0])
    T_r[0] = jnp.maximum(_mlp2(y, Wpa[0], bpa[0], Wpb[0], bpb[0]),
                         0.0).astype(jnp.bfloat16)


def _fin_body(z_r, st_r, Wua, Wub, bua, bub, out_r):
    out_r[...] = _masked_update(z_r[0], st_r[...],
                                Wua[0], Wub[0], bua[0], bub[0])


def _full2(shape):
    return pl.BlockSpec(shape, lambda c, i: (0, 0))


def _stk3(shape):
    return pl.BlockSpec(shape, lambda c, i: (c, 0, 0))


def _build_tc_calls(interpret=False):
    rows = pl.BlockSpec((_R, _D), lambda c, i: (i, 0))
    rows3 = pl.BlockSpec((1, _R, _D), lambda c, i: (c, i, 0))
    w = _full2((_D, _D))
    b = _full2((1, _D))
    w3 = _stk3((1, _D, _D))
    b3 = _stk3((1, 1, _D))

    init = pl.pallas_call(
        _init_body,
        grid=(_NC, _NBLK),
        in_specs=[rows, w, w, b, b, w3, w3, b3, b3],
        out_specs=[rows, rows3],
        out_shape=[jax.ShapeDtypeStruct((_N, _D), jnp.float32),
                   jax.ShapeDtypeStruct((_NC, _N, _D), jnp.bfloat16)],
        interpret=interpret,
    )
    mid = pl.pallas_call(
        _mid_body,
        grid=(_NC, _NBLK),
        in_specs=[rows3, rows, w3, w3, b3, b3, w3, w3, b3, b3],
        out_specs=rows3,
        out_shape=jax.ShapeDtypeStruct((_NC, _N, _D), jnp.bfloat16),
        interpret=interpret,
    )
    fin = pl.pallas_call(
        _fin_body,
        grid=(_NC, _NBLK),
        in_specs=[rows3, rows, w3, w3, b3, b3],
        out_specs=pl.BlockSpec((_R, _D), lambda c, i: (i, c)),
        out_shape=jax.ShapeDtypeStruct((_N, 2 * _D), jnp.float32),
        interpret=interpret,
    )
    return init, mid, fin


_init_call, _mid_call, _fin_call = _build_tc_calls()


def _sc_segment(T2, gidx, sidx, nb):
    mesh = plsc.VectorSubcoreMesh(core_axis_name="c", subcore_axis_name="s")
    nchunk = nb // _CHK
    zrows = _NZ // _NS

    @functools.partial(
        pl.kernel,
        out_type=jax.ShapeDtypeStruct((_NC, _NZ, _D), jnp.float32),
        mesh=mesh,
        compiler_params=pltpu.CompilerParams(use_tc_tiling_on_sc=False),
        scratch_types=[
            pltpu.VMEM((_CHK, _BATCH), jnp.int32),
            pltpu.VMEM((2 * _CHK, _HALF), jnp.int32),
            pltpu.VMEM((3, _BATCH, _D // 2), jnp.int32),
            pltpu.VMEM((2, _HALF, _D), jnp.float32),
            pltpu.VMEM_SHARED((_NZ, _D), jnp.float32),
            pltpu.SemaphoreType.DMA,
            pltpu.SemaphoreType.DMA,
        ],
    )
    def k(T_hbm, g_hbm, s_hbm, out_hbm, g_v, s_v, b16, f32b, z_sh,
          gsem, ssem):
        c = lax.axis_index("c")
        s = lax.axis_index("s")
        wid = c * _NS + s

        zbuf = f32b.at[0]

        def _zb(t, carry):
            zbuf[lax.div(t, 8), pl.ds(lax.rem(t, 8) * 16, 16)] = (
                jnp.zeros((16,), jnp.float32))
            return carry

        lax.fori_loop(0, _HALF * 8, _zb, 0)

        zb0 = s * zrows
        nfull = zrows // _HALF
        for j in range(nfull):
            pltpu.sync_copy(zbuf, z_sh.at[pl.ds(zb0 + j * _HALF, _HALF)])
        rem = zrows - nfull * _HALF
        if rem:
            pltpu.sync_copy(zbuf.at[pl.ds(0, rem)],
                            z_sh.at[pl.ds(zb0 + zrows - rem, rem)])

        plsc.subcore_barrier()

        for _ in range(2):
            pltpu.async_copy(zbuf, z_sh.at[pl.ds(_N, _HALF)], ssem)

        def _chunk(co, carry):
            pltpu.sync_copy(g_hbm.at[wid].at[pl.ds(co * _CHK, _CHK)], g_v)
            pltpu.sync_copy(
                s_hbm.at[wid].at[pl.ds(co * 2 * _CHK, 2 * _CHK)], s_v)
            for slot in range(3):
                pltpu.async_copy(
                    T_hbm.at[g_v.at[slot]], b16.at[slot], gsem)

            def _batch(bloc, inner):
                slot = lax.rem(bloc, 3)
                pltpu.make_async_copy(
                    T_hbm.at[pl.ds(0, _BATCH)], b16.at[slot],
                    gsem).wait()
                for h in range(2):
                    pltpu.make_async_copy(
                        out_hbm.at[c].at[pl.ds(0, _HALF)], f32b.at[h],
                        ssem).wait()

                    def _cv(t, carry2, _h=h):
                        r = _h * _HALF + lax.div(t, 4)
                        q = lax.rem(t, 4)
                        v = b16[slot, r, pl.ds(q * 16, 16)]
                        ev = lax.bitcast_convert_type(
                            lax.shift_left(v, 16), jnp.float32)
                        od = lax.bitcast_convert_type(
                            jnp.bitwise_and(v, jnp.int32(-65536)),
                            jnp.float32)
                        rr = lax.rem(r, _HALF)
                        f32b[_h, rr, pl.ds(q * 32, 16)] = ev
                        f32b[_h, rr, pl.ds(q * 32 + 16, 16)] = od
                        return carry2

                    lax.fori_loop(0, _HALF * 4, _cv, 0, unroll=8)
                    pltpu.async_copy(
                        f32b.at[h], z_sh.at[s_v.at[2 * bloc + h]],
                        ssem, add=True)

                @pl.when(bloc + 3 < _CHK)
                def _():
                    pltpu.async_copy(
                        T_hbm.at[g_v.at[bloc + 3]], b16.at[slot], gsem)

                return inner

            lax.fori_loop(0, _CHK, _batch, 0)
            return carry

        lax.fori_loop(0, nchunk, _chunk, 0)

        for h in range(2):
            pltpu.make_async_copy(
                out_hbm.at[c].at[pl.ds(0, _HALF)], f32b.at[h], ssem).wait()

        plsc.subcore_barrier()

        pltpu.sync_copy(z_sh.at[pl.ds(zb0, zrows)],
                        out_hbm.at[c].at[pl.ds(zb0, zrows)])

    return k(T2, gidx, sidx)


def _prep_indices(edge_index):
    E = edge_index.shape[1]
    nb = -(-E // (_NS * _BATCH * _CHK)) * _CHK
    cap = _NS * nb * _BATCH
    pad = cap - E

    src = edge_index[0].astype(jnp.int32)
    dst = edge_index[1].astype(jnp.int32)
    pz = jnp.zeros((pad,), jnp.int32)
    pr = jnp.full((pad,), _N, jnp.int32)
    gidx = jnp.concatenate([src, pz, dst + _N, pz]).reshape(
        _NC * _NS, nb, _BATCH)
    sidx = jnp.concatenate([dst, pr, src, pr]).reshape(
        _NC * _NS, 2 * nb, _HALF)
    return gidx, sidx, nb


def _prep_weights(Ws, bs):
    perm = jnp.asarray(_PERM)
    W0a, W0b = Ws[0, 0], Ws[0, 1]
    b0a = bs[0, 0].reshape(1, _D)
    b0b = bs[0, 1].reshape(1, _D)
    Wpa = jnp.stack([Ws[1, 0], Ws[3, 0]])
    Wpb = jnp.stack([Ws[1, 1], Ws[3, 1]])
    bpa = jnp.stack([bs[1, 0], bs[3, 0]])[:, None, :]
    bpb = jnp.stack([bs[1, 1], bs[3, 1]])[:, None, :]
    Wua = jnp.stack([Ws[2, 0], Ws[4, 0]])[:, perm, :]
    Wub = jnp.stack([Ws[2, 1], Ws[4, 1]])
    bua = jnp.stack([bs[2, 0], bs[4, 0]])[:, None, :]
    bub = jnp.stack([bs[2, 1], bs[4, 1]])[:, None, :]
    return (W0a, W0b, b0a, b0b, Wpa, Wpb, bpa, bpb,
            Wua, Wub, bua, bub)


def kernel(x, edge_index, Ws, bs):
    gidx, sidx, nb = _prep_indices(edge_index)
    (W0a, W0b, b0a, b0b, Wpa, Wpb, bpa, bpb,
     Wua, Wub, bua, bub) = _prep_weights(Ws, bs)

    st, T = _init_call(x, W0a, W0b, b0a, b0b, Wpa, Wpb, bpa, bpb)
    out = None
    for step in range(_K):
        Tp = lax.bitcast_convert_type(
            T.reshape(_NC * _N, _D // 2, 2), jnp.int32)
        zp = _sc_segment(Tp, gidx, sidx, nb)
        if step < _K - 1:
            T = _mid_call(zp, st, Wua, Wub, bua, bub, Wpa, Wpb, bpa, bpb)
        else:
            out = _fin_call(zp, st, Wua, Wub, bua, bub)
    return out


# speedup vs baseline: 1.7443x; 1.1499x over previous
import functools

import jax
import jax.numpy as jnp
import numpy as np
from jax import lax
from jax.experimental import pallas as pl
from jax.experimental.pallas import tpu as pltpu
from jax.experimental.pallas import tpu_sc as plsc

_N = 10000
_D = 128
_K = 3
_NC = 2
_NS = 16
_BATCH = 128
_HALF = 64
_CHK = 32
_NZ = _N + 112
_R = 2000
_NBLK = _N // _R
_GBYTES = _BATCH * (_D // 2) * 4
_SBYTES = _HALF * _D * 4

_PERM = np.concatenate([
    np.concatenate([16 * q + np.arange(16), 64 + 16 * q + np.arange(16)])
    for q in range(_D // 32)
])


def _mlp2(a, W0, b0, W1, b1):
    dn = (((1,), (0,)), ((), ()))
    h = lax.dot_general(a, W0, dn, precision=lax.Precision.HIGHEST,
                        preferred_element_type=jnp.float32) + b0
    h = jnp.maximum(h, 0.0)
    return lax.dot_general(h, W1, dn, precision=lax.Precision.HIGHEST,
                           preferred_element_type=jnp.float32) + b1


def _pack(t):
    t16 = t.astype(jnp.bfloat16)
    lo = lax.bitcast_convert_type(t16[:, :_D // 2],
                                  jnp.uint16).astype(jnp.int32)
    hi = lax.bitcast_convert_type(t16[:, _D // 2:],
                                  jnp.uint16).astype(jnp.int32)
    return jnp.bitwise_or(lo, lax.shift_left(hi, 16))


def _init_body(x_r, W0a, W0b, b0a, b0b, Wpa, Wpb, bpa, bpb, st_r, T_r):
    st = _mlp2(x_r[...], W0a[...], b0a[...], W0b[...], b0b[...])
    st_r[...] = st
    T_r[0] = _pack(jnp.maximum(
        _mlp2(st, Wpa[0], bpa[0], Wpb[0], bpb[0]), 0.0))


def _masked_update(z, st, Wua, Wub, bua, bub):
    c = pl.program_id(0)
    i = pl.program_id(1)
    u = jnp.maximum(_mlp2(z, Wua, bua, Wub, bub), 0.0)
    row = i * _R + lax.broadcasted_iota(jnp.int32, (_R, _D), 0)
    sink = jnp.where(c == 0, _N - 1, 0)
    return jnp.where(row == sink, 0.0, u) + st


def _mid_body(z_r, st_r, Wua, Wub, bua, bub, Wpa, Wpb, bpa, bpb, T_r):
    y = _masked_update(z_r[0], st_r[...], Wua[0], Wub[0], bua[0], bub[0])
    T_r[0] = _pack(jnp.maximum(
        _mlp2(y, Wpa[0], bpa[0], Wpb[0], bpb[0]), 0.0))


def _fin_body(z_r, st_r, Wua, Wub, bua, bub, out_r):
    out_r[...] = _masked_update(z_r[0], st_r[...],
                                Wua[0], Wub[0], bua[0], bub[0])


def _full2(shape):
    return pl.BlockSpec(shape, lambda c, i: (0, 0))


def _stk3(shape):
    return pl.BlockSpec(shape, lambda c, i: (c, 0, 0))


def _build_tc_calls(interpret=False):
    rows = pl.BlockSpec((_R, _D), lambda c, i: (i, 0))
    rows3 = pl.BlockSpec((1, _R, _D), lambda c, i: (c, i, 0))
    w = _full2((_D, _D))
    b = _full2((1, _D))
    w3 = _stk3((1, _D, _D))
    b3 = _stk3((1, 1, _D))

    init = pl.pallas_call(
        _init_body,
        grid=(_NC, _NBLK),
        in_specs=[rows, w, w, b, b, w3, w3, b3, b3],
        out_specs=[rows, pl.BlockSpec((1, _R, _D // 2),
                                      lambda c, i: (c, i, 0))],
        out_shape=[jax.ShapeDtypeStruct((_N, _D), jnp.float32),
                   jax.ShapeDtypeStruct((_NC, _N, _D // 2), jnp.int32)],
        interpret=interpret,
    )
    mid = pl.pallas_call(
        _mid_body,
        grid=(_NC, _NBLK),
        in_specs=[rows3, rows, w3, w3, b3, b3, w3, w3, b3, b3],
        out_specs=pl.BlockSpec((1, _R, _D // 2), lambda c, i: (c, i, 0)),
        out_shape=jax.ShapeDtypeStruct((_NC, _N, _D // 2), jnp.int32),
        interpret=interpret,
    )
    fin = pl.pallas_call(
        _fin_body,
        grid=(_NC, _NBLK),
        in_specs=[rows3, rows, w3, w3, b3, b3],
        out_specs=pl.BlockSpec((_R, _D), lambda c, i: (i, c)),
        out_shape=jax.ShapeDtypeStruct((_N, 2 * _D), jnp.float32),
        interpret=interpret,
    )
    return init, mid, fin


_init_call, _mid_call, _fin_call = _build_tc_calls()


def _sc_segment(T2, gidx, sidx, nb):
    mesh = plsc.VectorSubcoreMesh(core_axis_name="c", subcore_axis_name="s")
    nchunk = nb // _CHK
    zrows = _NZ // _NS

    @functools.partial(
        pl.kernel,
        out_type=jax.ShapeDtypeStruct((_NC, _NZ, _D), jnp.float32),
        mesh=mesh,
        compiler_params=pltpu.CompilerParams(use_tc_tiling_on_sc=False),
        scratch_types=[
            pltpu.VMEM((_CHK, _BATCH), jnp.int32),
            pltpu.VMEM((2 * _CHK, _HALF), jnp.int32),
            pltpu.VMEM((2, _BATCH, _D // 2), jnp.int32),
            pltpu.VMEM((2, _HALF, _D), jnp.float32),
            pltpu.VMEM_SHARED((_NZ, _D), jnp.float32),
            pltpu.SemaphoreType.DMA,
            pltpu.SemaphoreType.DMA,
        ],
    )
    def k(T_hbm, g_hbm, s_hbm, out_hbm, g_v, s_v, b16, f32b, z_sh,
          gsem, ssem):
        c = lax.axis_index("c")
        s = lax.axis_index("s")
        wid = c * _NS + s

        zbuf = f32b.at[0]

        def _zb(t, carry):
            zbuf[lax.div(t, 8), pl.ds(lax.rem(t, 8) * 16, 16)] = (
                jnp.zeros((16,), jnp.float32))
            return carry

        lax.fori_loop(0, _HALF * 8, _zb, 0)

        zb0 = s * zrows
        nfull = zrows // _HALF
        for j in range(nfull):
            pltpu.sync_copy(zbuf, z_sh.at[pl.ds(zb0 + j * _HALF, _HALF)])
        rem = zrows - nfull * _HALF
        if rem:
            pltpu.sync_copy(zbuf.at[pl.ds(0, rem)],
                            z_sh.at[pl.ds(zb0 + zrows - rem, rem)])

        plsc.subcore_barrier()

        for _ in range(2):
            pltpu.async_copy(zbuf, z_sh.at[pl.ds(_N, _HALF)], ssem)

        def _chunk(co, carry):
            pltpu.sync_copy(g_hbm.at[wid].at[pl.ds(co * _CHK, _CHK)], g_v)
            pltpu.sync_copy(
                s_hbm.at[wid].at[pl.ds(co * 2 * _CHK, 2 * _CHK)], s_v)
            for slot in range(2):
                pltpu.async_copy(
                    T_hbm.at[g_v.at[slot]], b16.at[slot], gsem)

            def _pair(p, inner):
                for slot in range(2):
                    bloc = 2 * p + slot
                    pltpu.make_async_copy(
                        T_hbm.at[pl.ds(0, _BATCH)], b16.at[slot],
                        gsem).wait()
                    for h in range(2):
                        pltpu.make_async_copy(
                            out_hbm.at[c].at[pl.ds(0, _HALF)], f32b.at[h],
                            ssem).wait()

                        def _cv(t, carry2, _slot=slot, _h=h):
                            r = _h * _HALF + lax.div(t, 4)
                            q = lax.rem(t, 4)
                            v = b16[_slot, r, pl.ds(q * 16, 16)]
                            ev = lax.bitcast_convert_type(
                                lax.shift_left(v, 16), jnp.float32)
                            od = lax.bitcast_convert_type(
                                jnp.bitwise_and(v, jnp.int32(-65536)),
                                jnp.float32)
                            rr = lax.rem(r, _HALF)
                            f32b[_h, rr, pl.ds(q * 32, 16)] = ev
                            f32b[_h, rr, pl.ds(q * 32 + 16, 16)] = od
                            return carry2

                        lax.fori_loop(0, _HALF * 4, _cv, 0, unroll=8)
                        pltpu.async_copy(
                            f32b.at[h], z_sh.at[s_v.at[2 * bloc + h]],
                            ssem, add=True)

                    @pl.when(p < _CHK // 2 - 1)
                    def _():
                        pltpu.async_copy(
                            T_hbm.at[g_v.at[bloc + 2]], b16.at[slot], gsem)

                return inner

            lax.fori_loop(0, _CHK // 2, _pair, 0)
            return carry

        lax.fori_loop(0, nchunk, _chunk, 0)

        for h in range(2):
            pltpu.make_async_copy(
                out_hbm.at[c].at[pl.ds(0, _HALF)], f32b.at[h], ssem).wait()

        plsc.subcore_barrier()

        pltpu.sync_copy(z_sh.at[pl.ds(zb0, zrows)],
                        out_hbm.at[c].at[pl.ds(zb0, zrows)])

    return k(T2, gidx, sidx)


def _prep_indices(edge_index):
    E = edge_index.shape[1]
    nb = -(-E // (_NS * _BATCH * _CHK)) * _CHK
    cap = _NS * nb * _BATCH
    pad = cap - E

    src = edge_index[0].astype(jnp.int32)
    dst = edge_index[1].astype(jnp.int32)
    pz = jnp.zeros((pad,), jnp.int32)
    pr = jnp.full((pad,), _N, jnp.int32)
    gidx = jnp.concatenate([src, pz, dst + _N, pz]).reshape(
        _NC * _NS, nb, _BATCH)
    sidx = jnp.concatenate([dst, pr, src, pr]).reshape(
        _NC * _NS, 2 * nb, _HALF)
    return gidx, sidx, nb


def _prep_weights(Ws, bs):
    perm = jnp.asarray(_PERM)
    W0a, W0b = Ws[0, 0], Ws[0, 1]
    b0a = bs[0, 0].reshape(1, _D)
    b0b = bs[0, 1].reshape(1, _D)
    Wpa = jnp.stack([Ws[1, 0], Ws[3, 0]])
    Wpb = jnp.stack([Ws[1, 1], Ws[3, 1]])
    bpa = jnp.stack([bs[1, 0], bs[3, 0]])[:, None, :]
    bpb = jnp.stack([bs[1, 1], bs[3, 1]])[:, None, :]
    Wua = jnp.stack([Ws[2, 0], Ws[4, 0]])[:, perm, :]
    Wub = jnp.stack([Ws[2, 1], Ws[4, 1]])
    bua = jnp.stack([bs[2, 0], bs[4, 0]])[:, None, :]
    bub = jnp.stack([bs[2, 1], bs[4, 1]])[:, None, :]
    return (W0a, W0b, b0a, b0b, Wpa, Wpb, bpa, bpb,
            Wua, Wub, bua, bub)


def kernel(x, edge_index, Ws, bs):
    gidx, sidx, nb = _prep_indices(edge_index)
    (W0a, W0b, b0a, b0b, Wpa, Wpb, bpa, bpb,
     Wua, Wub, bua, bub) = _prep_weights(Ws, bs)

    st, T = _init_call(x, W0a, W0b, b0a, b0b, Wpa, Wpb, bpa, bpb)
    out = None
    for step in range(_K):
        zp = _sc_segment(T.reshape(_NC * _N, _D // 2), gidx, sidx, nb)
        if step < _K - 1:
            T = _mid_call(zp, st, Wua, Wub, bua, bub, Wpa, Wpb, bpa, bpb)
        else:
            out = _fin_call(zp, st, Wua, Wub, bua, bub)
    return out


# speedup vs baseline: 1.7478x; 1.0020x over previous
import functools

import jax
import jax.numpy as jnp
import numpy as np
from jax import lax
from jax.experimental import pallas as pl
from jax.experimental.pallas import tpu as pltpu
from jax.experimental.pallas import tpu_sc as plsc

_N = 10000
_D = 128
_K = 3
_NC = 2
_NS = 16
_BATCH = 128
_HALF = 64
_CHK = 32
_NZ = _N + 112
_R = 2000
_NBLK = _N // _R
_GBYTES = _BATCH * (_D // 2) * 4
_SBYTES = _HALF * _D * 4

_PERM = np.concatenate([
    np.concatenate([16 * q + np.arange(16), 64 + 16 * q + np.arange(16)])
    for q in range(_D // 32)
])


def _mlp2(a, W0, b0, W1, b1):
    dn = (((1,), (0,)), ((), ()))
    h = lax.dot_general(a, W0, dn, precision=lax.Precision.HIGHEST,
                        preferred_element_type=jnp.float32) + b0
    h = jnp.maximum(h, 0.0)
    return lax.dot_general(h, W1, dn, precision=lax.Precision.HIGHEST,
                           preferred_element_type=jnp.float32) + b1


def _pack(t):
    t16 = t.astype(jnp.bfloat16)
    lo = lax.bitcast_convert_type(t16[:, :_D // 2],
                                  jnp.uint16).astype(jnp.int32)
    hi = lax.bitcast_convert_type(t16[:, _D // 2:],
                                  jnp.uint16).astype(jnp.int32)
    return jnp.bitwise_or(lo, lax.shift_left(hi, 16))


def _init_body(x_r, W0a, W0b, b0a, b0b, Wpa, Wpb, bpa, bpb, st_r, T_r):
    st = _mlp2(x_r[...], W0a[...], b0a[...], W0b[...], b0b[...])
    st_r[...] = st
    T_r[0] = _pack(jnp.maximum(
        _mlp2(st, Wpa[0], bpa[0], Wpb[0], bpb[0]), 0.0))


def _masked_update(z, st, Wua, Wub, bua, bub):
    c = pl.program_id(0)
    i = pl.program_id(1)
    u = jnp.maximum(_mlp2(z, Wua, bua, Wub, bub), 0.0)
    row = i * _R + lax.broadcasted_iota(jnp.int32, (_R, _D), 0)
    sink = jnp.where(c == 0, _N - 1, 0)
    return jnp.where(row == sink, 0.0, u) + st


def _mid_body(z_r, st_r, Wua, Wub, bua, bub, Wpa, Wpb, bpa, bpb, T_r):
    y = _masked_update(z_r[0], st_r[...], Wua[0], Wub[0], bua[0], bub[0])
    T_r[0] = _pack(jnp.maximum(
        _mlp2(y, Wpa[0], bpa[0], Wpb[0], bpb[0]), 0.0))


def _fin_body(z_r, st_r, Wua, Wub, bua, bub, out_r):
    out_r[...] = _masked_update(z_r[0], st_r[...],
                                Wua[0], Wub[0], bua[0], bub[0])


def _full2(shape):
    return pl.BlockSpec(shape, lambda c, i: (0, 0))


def _stk3(shape):
    return pl.BlockSpec(shape, lambda c, i: (c, 0, 0))


def _build_tc_calls(interpret=False):
    rows = pl.BlockSpec((_R, _D), lambda c, i: (i, 0))
    rows3 = pl.BlockSpec((1, _R, _D), lambda c, i: (c, i, 0))
    w = _full2((_D, _D))
    b = _full2((1, _D))
    w3 = _stk3((1, _D, _D))
    b3 = _stk3((1, 1, _D))

    init = pl.pallas_call(
        _init_body,
        grid=(_NC, _NBLK),
        in_specs=[rows, w, w, b, b, w3, w3, b3, b3],
        out_specs=[rows, pl.BlockSpec((1, _R, _D // 2),
                                      lambda c, i: (c, i, 0))],
        out_shape=[jax.ShapeDtypeStruct((_N, _D), jnp.float32),
                   jax.ShapeDtypeStruct((_NC, _N, _D // 2), jnp.int32)],
        interpret=interpret,
    )
    mid = pl.pallas_call(
        _mid_body,
        grid=(_NC, _NBLK),
        in_specs=[rows3, rows, w3, w3, b3, b3, w3, w3, b3, b3],
        out_specs=pl.BlockSpec((1, _R, _D // 2), lambda c, i: (c, i, 0)),
        out_shape=jax.ShapeDtypeStruct((_NC, _N, _D // 2), jnp.int32),
        interpret=interpret,
    )
    fin = pl.pallas_call(
        _fin_body,
        grid=(_NC, _NBLK),
        in_specs=[rows3, rows, w3, w3, b3, b3],
        out_specs=pl.BlockSpec((_R, _D), lambda c, i: (i, c)),
        out_shape=jax.ShapeDtypeStruct((_N, 2 * _D), jnp.float32),
        interpret=interpret,
    )
    return init, mid, fin


_init_call, _mid_call, _fin_call = _build_tc_calls()


def _sc_segment(T2, gidx, sidx, nb):
    mesh = plsc.VectorSubcoreMesh(core_axis_name="c", subcore_axis_name="s")
    nchunk = nb // _CHK
    zrows = _NZ // _NS

    @functools.partial(
        pl.kernel,
        out_type=jax.ShapeDtypeStruct((_NC, _NZ, _D), jnp.float32),
        mesh=mesh,
        compiler_params=pltpu.CompilerParams(use_tc_tiling_on_sc=False),
        scratch_types=[
            pltpu.VMEM((_CHK, _BATCH), jnp.int32),
            pltpu.VMEM((2 * _CHK, _HALF), jnp.int32),
            pltpu.VMEM((2, _BATCH, _D // 2), jnp.int32),
            pltpu.VMEM((2, _HALF, _D), jnp.float32),
            pltpu.VMEM_SHARED((_NZ, _D), jnp.float32),
            pltpu.SemaphoreType.DMA,
            pltpu.SemaphoreType.DMA,
        ],
    )
    def k(T_hbm, g_hbm, s_hbm, out_hbm, g_v, s_v, b16, f32b, z_sh,
          gsem, ssem):
        c = lax.axis_index("c")
        s = lax.axis_index("s")
        wid = c * _NS + s

        zbuf = f32b.at[0]

        def _zb(t, carry):
            zbuf[lax.div(t, 8), pl.ds(lax.rem(t, 8) * 16, 16)] = (
                jnp.zeros((16,), jnp.float32))
            return carry

        lax.fori_loop(0, _HALF * 8, _zb, 0)

        zb0 = s * zrows
        nfull = zrows // _HALF
        for j in range(nfull):
            pltpu.sync_copy(zbuf, z_sh.at[pl.ds(zb0 + j * _HALF, _HALF)])
        rem = zrows - nfull * _HALF
        if rem:
            pltpu.sync_copy(zbuf.at[pl.ds(0, rem)],
                            z_sh.at[pl.ds(zb0 + zrows - rem, rem)])

        plsc.subcore_barrier()

        for _ in range(2):
            pltpu.async_copy(zbuf, z_sh.at[pl.ds(_N, _HALF)], ssem)

        def _chunk(co, carry):
            pltpu.sync_copy(g_hbm.at[wid].at[pl.ds(co * _CHK, _CHK)], g_v)
            pltpu.sync_copy(
                s_hbm.at[wid].at[pl.ds(co * 2 * _CHK, 2 * _CHK)], s_v)
            for slot in range(2):
                pltpu.async_copy(
                    T_hbm.at[g_v.at[slot]], b16.at[slot], gsem)

            def _pair(p, inner):
                for slot in range(2):
                    bloc = 2 * p + slot
                    pltpu.make_async_copy(
                        T_hbm.at[pl.ds(0, _BATCH)], b16.at[slot],
                        gsem).wait()
                    for h in range(2):
                        pltpu.make_async_copy(
                            out_hbm.at[c].at[pl.ds(0, _HALF)], f32b.at[h],
                            ssem).wait()

                        def _cv(r, carry2, _slot=slot, _h=h):
                            row = _h * _HALF + r
                            for q in range(4):
                                v = b16[_slot, row, pl.ds(q * 16, 16)]
                                ev = lax.bitcast_convert_type(
                                    lax.shift_left(v, 16), jnp.float32)
                                od = lax.bitcast_convert_type(
                                    jnp.bitwise_and(v, jnp.int32(-65536)),
                                    jnp.float32)
                                f32b[_h, r, pl.ds(q * 32, 16)] = ev
                                f32b[_h, r, pl.ds(q * 32 + 16, 16)] = od
                            return carry2

                        lax.fori_loop(0, _HALF, _cv, 0, unroll=4)
                        pltpu.async_copy(
                            f32b.at[h], z_sh.at[s_v.at[2 * bloc + h]],
                            ssem, add=True)

                    @pl.when(p < _CHK // 2 - 1)
                    def _():
                        pltpu.async_copy(
                            T_hbm.at[g_v.at[bloc + 2]], b16.at[slot], gsem)

                return inner

            lax.fori_loop(0, _CHK // 2, _pair, 0)
            return carry

        lax.fori_loop(0, nchunk, _chunk, 0)

        for h in range(2):
            pltpu.make_async_copy(
                out_hbm.at[c].at[pl.ds(0, _HALF)], f32b.at[h], ssem).wait()

        plsc.subcore_barrier()

        pltpu.sync_copy(z_sh.at[pl.ds(zb0, zrows)],
                        out_hbm.at[c].at[pl.ds(zb0, zrows)])

    return k(T2, gidx, sidx)


def _prep_indices(edge_index):
    E = edge_index.shape[1]
    nb = -(-E // (_NS * _BATCH * _CHK)) * _CHK
    cap = _NS * nb * _BATCH
    pad = cap - E

    src = edge_index[0].astype(jnp.int32)
    dst = edge_index[1].astype(jnp.int32)
    pz = jnp.zeros((pad,), jnp.int32)
    pr = jnp.full((pad,), _N, jnp.int32)
    gidx = jnp.concatenate([src, pz, dst + _N, pz]).reshape(
        _NC * _NS, nb, _BATCH)
    sidx = jnp.concatenate([dst, pr, src, pr]).reshape(
        _NC * _NS, 2 * nb, _HALF)
    return gidx, sidx, nb


def _prep_weights(Ws, bs):
    perm = jnp.asarray(_PERM)
    W0a, W0b = Ws[0, 0], Ws[0, 1]
    b0a = bs[0, 0].reshape(1, _D)
    b0b = bs[0, 1].reshape(1, _D)
    Wpa = jnp.stack([Ws[1, 0], Ws[3, 0]])
    Wpb = jnp.stack([Ws[1, 1], Ws[3, 1]])
    bpa = jnp.stack([bs[1, 0], bs[3, 0]])[:, None, :]
    bpb = jnp.stack([bs[1, 1], bs[3, 1]])[:, None, :]
    Wua = jnp.stack([Ws[2, 0], Ws[4, 0]])[:, perm, :]
    Wub = jnp.stack([Ws[2, 1], Ws[4, 1]])
    bua = jnp.stack([bs[2, 0], bs[4, 0]])[:, None, :]
    bub = jnp.stack([bs[2, 1], bs[4, 1]])[:, None, :]
    return (W0a, W0b, b0a, b0b, Wpa, Wpb, bpa, bpb,
            Wua, Wub, bua, bub)


def kernel(x, edge_index, Ws, bs):
    gidx, sidx, nb = _prep_indices(edge_index)
    (W0a, W0b, b0a, b0b, Wpa, Wpb, bpa, bpb,
     Wua, Wub, bua, bub) = _prep_weights(Ws, bs)

    st, T = _init_call(x, W0a, W0b, b0a, b0b, Wpa, Wpb, bpa, bpb)
    out = None
    for step in range(_K):
        zp = _sc_segment(T.reshape(_NC * _N, _D // 2), gidx, sidx, nb)
        if step < _K - 1:
            T = _mid_call(zp, st, Wua, Wub, bua, bub, Wpa, Wpb, bpa, bpb)
        else:
            out = _fin_call(zp, st, Wua, Wub, bua, bub)
    return out


# speedup vs baseline: 1.7648x; 1.0097x over previous
import functools

import jax
import jax.numpy as jnp
import numpy as np
from jax import lax
from jax.experimental import pallas as pl
from jax.experimental.pallas import tpu as pltpu
from jax.experimental.pallas import tpu_sc as plsc

_N = 10000
_D = 128
_K = 3
_NC = 2
_NS = 16
_BATCH = 128
_HALF = 64
_CHK = 32
_NZ = _N + 112
_R = 2000
_NBLK = _N // _R
_GBYTES = _BATCH * (_D // 2) * 4
_SBYTES = _HALF * _D * 4

_PERM = np.concatenate([
    np.concatenate([16 * q + np.arange(16), 64 + 16 * q + np.arange(16)])
    for q in range(_D // 32)
])


def _mlp2(a, W0, b0, W1, b1):
    dn = (((1,), (0,)), ((), ()))
    h = lax.dot_general(a, W0, dn, precision=lax.Precision.HIGHEST,
                        preferred_element_type=jnp.float32) + b0
    h = jnp.maximum(h, 0.0)
    return lax.dot_general(h, W1, dn, precision=lax.Precision.HIGHEST,
                           preferred_element_type=jnp.float32) + b1


def _pack(t):
    t16 = t.astype(jnp.bfloat16)
    lo = lax.bitcast_convert_type(t16[:, :_D // 2],
                                  jnp.uint16).astype(jnp.int32)
    hi = lax.bitcast_convert_type(t16[:, _D // 2:],
                                  jnp.uint16).astype(jnp.int32)
    return jnp.bitwise_or(lo, lax.shift_left(hi, 16))


def _init_body(x_r, W0a, W0b, b0a, b0b, Wpa, Wpb, bpa, bpb, st_r, T_r):
    st = _mlp2(x_r[...], W0a[...], b0a[...], W0b[...], b0b[...])
    st_r[...] = st
    T_r[0] = _pack(jnp.maximum(
        _mlp2(st, Wpa[0], bpa[0], Wpb[0], bpb[0]), 0.0))


def _masked_update(z, st, Wua, Wub, bua, bub):
    c = pl.program_id(0)
    i = pl.program_id(1)
    u = jnp.maximum(_mlp2(z, Wua, bua, Wub, bub), 0.0)
    row = i * _R + lax.broadcasted_iota(jnp.int32, (_R, _D), 0)
    sink = jnp.where(c == 0, _N - 1, 0)
    return jnp.where(row == sink, 0.0, u) + st


def _mid_body(z_r, st_r, Wua, Wub, bua, bub, Wpa, Wpb, bpa, bpb, T_r):
    y = _masked_update(z_r[0], st_r[...], Wua[0], Wub[0], bua[0], bub[0])
    T_r[0] = _pack(jnp.maximum(
        _mlp2(y, Wpa[0], bpa[0], Wpb[0], bpb[0]), 0.0))


def _fin_body(z_r, st_r, Wua, Wub, bua, bub, out_r):
    out_r[...] = _masked_update(z_r[0], st_r[...],
                                Wua[0], Wub[0], bua[0], bub[0])


def _full2(shape):
    return pl.BlockSpec(shape, lambda c, i: (0, 0))


def _stk3(shape):
    return pl.BlockSpec(shape, lambda c, i: (c, 0, 0))


def _build_tc_calls(interpret=False):
    rows = pl.BlockSpec((_R, _D), lambda c, i: (i, 0))
    rows3 = pl.BlockSpec((1, _R, _D), lambda c, i: (c, i, 0))
    w = _full2((_D, _D))
    b = _full2((1, _D))
    w3 = _stk3((1, _D, _D))
    b3 = _stk3((1, 1, _D))

    init = pl.pallas_call(
        _init_body,
        grid=(_NC, _NBLK),
        in_specs=[rows, w, w, b, b, w3, w3, b3, b3],
        out_specs=[rows, pl.BlockSpec((1, _R, _D // 2),
                                      lambda c, i: (c, i, 0))],
        out_shape=[jax.ShapeDtypeStruct((_N, _D), jnp.float32),
                   jax.ShapeDtypeStruct((_NC, _N, _D // 2), jnp.int32)],
        interpret=interpret,
    )
    mid = pl.pallas_call(
        _mid_body,
        grid=(_NC, _NBLK),
        in_specs=[rows3, rows, w3, w3, b3, b3, w3, w3, b3, b3],
        out_specs=pl.BlockSpec((1, _R, _D // 2), lambda c, i: (c, i, 0)),
        out_shape=jax.ShapeDtypeStruct((_NC, _N, _D // 2), jnp.int32),
        interpret=interpret,
    )
    fin = pl.pallas_call(
        _fin_body,
        grid=(_NC, _NBLK),
        in_specs=[rows3, rows, w3, w3, b3, b3],
        out_specs=pl.BlockSpec((_R, _D), lambda c, i: (i, c)),
        out_shape=jax.ShapeDtypeStruct((_N, 2 * _D), jnp.float32),
        interpret=interpret,
    )
    return init, mid, fin


_init_call, _mid_call, _fin_call = _build_tc_calls()


def _sc_segment(T2, gidx, sidx, nb):
    mesh = plsc.VectorSubcoreMesh(core_axis_name="c", subcore_axis_name="s")
    nchunk = nb // _CHK
    zrows = _NZ // _NS

    @functools.partial(
        pl.kernel,
        out_type=jax.ShapeDtypeStruct((_NC, _NZ, _D), jnp.float32),
        mesh=mesh,
        compiler_params=pltpu.CompilerParams(use_tc_tiling_on_sc=False),
        scratch_types=[
            pltpu.VMEM((2, _CHK, _BATCH), jnp.int32),
            pltpu.VMEM((2, 2 * _CHK, _HALF), jnp.int32),
            pltpu.VMEM((2, _BATCH, _D // 2), jnp.int32),
            pltpu.VMEM((2, _HALF, _D), jnp.float32),
            pltpu.VMEM_SHARED((_NZ, _D), jnp.float32),
            pltpu.SemaphoreType.DMA,
            pltpu.SemaphoreType.DMA,
            pltpu.SemaphoreType.DMA,
        ],
    )
    def k(T_hbm, g_hbm, s_hbm, out_hbm, g_v, s_v, b16, f32b, z_sh,
          gsem, ssem, isem):
        c = lax.axis_index("c")
        s = lax.axis_index("s")
        wid = c * _NS + s

        zbuf = f32b.at[0]

        def _zb(t, carry):
            zbuf[lax.div(t, 8), pl.ds(lax.rem(t, 8) * 16, 16)] = (
                jnp.zeros((16,), jnp.float32))
            return carry

        lax.fori_loop(0, _HALF * 8, _zb, 0)

        pltpu.async_copy(g_hbm.at[wid].at[pl.ds(0, _CHK)], g_v.at[0], isem)
        pltpu.async_copy(s_hbm.at[wid].at[pl.ds(0, 2 * _CHK)], s_v.at[0],
                         isem)

        zb0 = s * zrows
        nfull = zrows // _HALF
        zcp = []
        for j in range(nfull):
            zcp.append(pltpu.async_copy(
                zbuf, z_sh.at[pl.ds(zb0 + j * _HALF, _HALF)], gsem))
        rem = zrows - nfull * _HALF
        if rem:
            zcp.append(pltpu.async_copy(
                zbuf.at[pl.ds(0, rem)],
                z_sh.at[pl.ds(zb0 + zrows - rem, rem)], gsem))
        for cp in zcp:
            cp.wait()

        plsc.subcore_barrier()

        for _ in range(2):
            pltpu.async_copy(zbuf, z_sh.at[pl.ds(_N, _HALF)], ssem)

        def _chunk(co, carry):
            par = lax.rem(co, 2)
            pltpu.make_async_copy(
                g_hbm.at[wid].at[pl.ds(0, _CHK)], g_v.at[par], isem).wait()
            pltpu.make_async_copy(
                s_hbm.at[wid].at[pl.ds(0, 2 * _CHK)], s_v.at[par],
                isem).wait()

            @pl.when(co + 1 < nchunk)
            def _():
                nxt = 1 - par
                pltpu.async_copy(
                    g_hbm.at[wid].at[pl.ds((co + 1) * _CHK, _CHK)],
                    g_v.at[nxt], isem)
                pltpu.async_copy(
                    s_hbm.at[wid].at[pl.ds((co + 1) * 2 * _CHK, 2 * _CHK)],
                    s_v.at[nxt], isem)

            gv = g_v.at[par]
            sv = s_v.at[par]
            for slot in range(2):
                pltpu.async_copy(
                    T_hbm.at[gv.at[slot]], b16.at[slot], gsem)

            def _pair(p, inner):
                for slot in range(2):
                    bloc = 2 * p + slot
                    pltpu.make_async_copy(
                        T_hbm.at[pl.ds(0, _BATCH)], b16.at[slot],
                        gsem).wait()
                    for h in range(2):
                        pltpu.make_async_copy(
                            out_hbm.at[c].at[pl.ds(0, _HALF)], f32b.at[h],
                            ssem).wait()

                        def _cv(r, carry2, _slot=slot, _h=h):
                            row = _h * _HALF + r
                            for q in range(4):
                                v = b16[_slot, row, pl.ds(q * 16, 16)]
                                ev = lax.bitcast_convert_type(
                                    lax.shift_left(v, 16), jnp.float32)
                                od = lax.bitcast_convert_type(
                                    jnp.bitwise_and(v, jnp.int32(-65536)),
                                    jnp.float32)
                                f32b[_h, r, pl.ds(q * 32, 16)] = ev
                                f32b[_h, r, pl.ds(q * 32 + 16, 16)] = od
                            return carry2

                        lax.fori_loop(0, _HALF, _cv, 0, unroll=4)
                        pltpu.async_copy(
                            f32b.at[h], z_sh.at[sv.at[2 * bloc + h]],
                            ssem, add=True)

                    @pl.when(p < _CHK // 2 - 1)
                    def _():
                        pltpu.async_copy(
                            T_hbm.at[gv.at[bloc + 2]], b16.at[slot], gsem)

                return inner

            lax.fori_loop(0, _CHK // 2, _pair, 0)
            return carry

        lax.fori_loop(0, nchunk, _chunk, 0)

        for h in range(2):
            pltpu.make_async_copy(
                out_hbm.at[c].at[pl.ds(0, _HALF)], f32b.at[h], ssem).wait()

        plsc.subcore_barrier()

        pltpu.sync_copy(z_sh.at[pl.ds(zb0, zrows)],
                        out_hbm.at[c].at[pl.ds(zb0, zrows)])

    return k(T2, gidx, sidx)


def _prep_indices(edge_index):
    E = edge_index.shape[1]
    nb = -(-E // (_NS * _BATCH * _CHK)) * _CHK
    cap = _NS * nb * _BATCH
    pad = cap - E

    src = edge_index[0].astype(jnp.int32)
    dst = edge_index[1].astype(jnp.int32)
    pz = jnp.zeros((pad,), jnp.int32)
    pr = jnp.full((pad,), _N, jnp.int32)
    gidx = jnp.concatenate([src, pz, dst + _N, pz]).reshape(
        _NC * _NS, nb, _BATCH)
    sidx = jnp.concatenate([dst, pr, src, pr]).reshape(
        _NC * _NS, 2 * nb, _HALF)
    return gidx, sidx, nb


def _prep_weights(Ws, bs):
    perm = jnp.asarray(_PERM)
    W0a, W0b = Ws[0, 0], Ws[0, 1]
    b0a = bs[0, 0].reshape(1, _D)
    b0b = bs[0, 1].reshape(1, _D)
    Wpa = jnp.stack([Ws[1, 0], Ws[3, 0]])
    Wpb = jnp.stack([Ws[1, 1], Ws[3, 1]])
    bpa = jnp.stack([bs[1, 0], bs[3, 0]])[:, None, :]
    bpb = jnp.stack([bs[1, 1], bs[3, 1]])[:, None, :]
    Wua = jnp.stack([Ws[2, 0], Ws[4, 0]])[:, perm, :]
    Wub = jnp.stack([Ws[2, 1], Ws[4, 1]])
    bua = jnp.stack([bs[2, 0], bs[4, 0]])[:, None, :]
    bub = jnp.stack([bs[2, 1], bs[4, 1]])[:, None, :]
    return (W0a, W0b, b0a, b0b, Wpa, Wpb, bpa, bpb,
            Wua, Wub, bua, bub)


def kernel(x, edge_index, Ws, bs):
    gidx, sidx, nb = _prep_indices(edge_index)
    (W0a, W0b, b0a, b0b, Wpa, Wpb, bpa, bpb,
     Wua, Wub, bua, bub) = _prep_weights(Ws, bs)

    st, T = _init_call(x, W0a, W0b, b0a, b0b, Wpa, Wpb, bpa, bpb)
    out = None
    for step in range(_K):
        zp = _sc_segment(T.reshape(_NC * _N, _D // 2), gidx, sidx, nb)
        if step < _K - 1:
            T = _mid_call(zp, st, Wua, Wub, bua, bub, Wpa, Wpb, bpa, bpb)
        else:
            out = _fin_call(zp, st, Wua, Wub, bua, bub)
    return out
